# baseline (speedup 1.0000x reference)
import functools
import jax
import jax.numpy as jnp
from jax import lax
from jax.experimental import pallas as pl
from jax.experimental.pallas import tpu as pltpu
from jax.experimental.pallas import tpu_sc as plsc

N = 10000
D = 128
H = 128
E = 320000

_NC = 2
_NS = 16
_NW = _NC * _NS
_L = 16

_EPT = E // _NW
_EPAD = 10240
_NB1 = 5
_G1 = _EPT // _L
_T1 = _G1 // _NB1
_NB2 = 4
_G2 = _EPAD // _L
_T2 = _G2 // _NB2
_RPT = 624
_RCH = 104

_BM = 1000

_SC_PARAMS = pltpu.CompilerParams(needs_layout_passes=False)
_SC_MESH = plsc.VectorSubcoreMesh(core_axis_name="c", subcore_axis_name="s")


def _mm4_body(x_ref, wz_ref, wi_ref, u_ref, z_ref, zi_ref, a_ref):
    x = x_ref[...]
    z_ref[...] = jnp.dot(x, wz_ref[...], preferred_element_type=jnp.float32)
    zi_ref[...] = jnp.dot(x, wi_ref[...], preferred_element_type=jnp.float32)
    a_ref[...] = jnp.dot(x, u_ref[...], preferred_element_type=jnp.float32)


_MM4_OUT = [
    jax.ShapeDtypeStruct((N, H), jnp.float32),
    jax.ShapeDtypeStruct((N, H), jnp.float32),
    jax.ShapeDtypeStruct((N, 2), jnp.float32),
]
_MM4_OUT_SPECS = [
    pl.BlockSpec((_BM, H), lambda i: (i, 0)),
    pl.BlockSpec((_BM, H), lambda i: (i, 0)),
    pl.BlockSpec((_BM, 2), lambda i: (i, 0)),
]


def _mm4(x, wz, wi, u):
    k = x.shape[1]
    return pl.pallas_call(
        _mm4_body,
        grid=(N // _BM,),
        in_specs=[
            pl.BlockSpec((_BM, k), lambda i: (i, 0)),
            pl.BlockSpec((k, H), lambda i: (0, 0)),
            pl.BlockSpec((k, H), lambda i: (0, 0)),
            pl.BlockSpec((k, 2), lambda i: (0, 0)),
        ],
        out_specs=_MM4_OUT_SPECS,
        out_shape=_MM4_OUT,
    )(x, wz, wi, u)


def _combine_mm4_body(zi_ref, sp_ref, dp_ref, wz_ref, wi_ref, u_ref,
                      z_ref, zo_ref, a_ref):
    den = dp_ref[:, 0:1] + dp_ref[:, 1:2]
    den = jnp.where(den > 0, den, 1.0)
    h = jnp.maximum(zi_ref[...] + (sp_ref[0] + sp_ref[1]) / den, 0.0)
    z_ref[...] = jnp.dot(h, wz_ref[...], preferred_element_type=jnp.float32)
    zo_ref[...] = jnp.dot(h, wi_ref[...], preferred_element_type=jnp.float32)
    a_ref[...] = jnp.dot(h, u_ref[...], preferred_element_type=jnp.float32)


def _combine_mm4(zi, sp, dp, wz, wi, u):
    return pl.pallas_call(
        _combine_mm4_body,
        grid=(N // _BM,),
        in_specs=[
            pl.BlockSpec((_BM, H), lambda i: (i, 0)),
            pl.BlockSpec((2, _BM, H), lambda i: (0, i, 0)),
            pl.BlockSpec((_BM, 2), lambda i: (i, 0)),
            pl.BlockSpec((H, H), lambda i: (0, 0)),
            pl.BlockSpec((H, H), lambda i: (0, 0)),
            pl.BlockSpec((H, 2), lambda i: (0, 0)),
        ],
        out_specs=_MM4_OUT_SPECS,
        out_shape=_MM4_OUT,
    )(zi, sp, dp, wz, wi, u)


def _combine_body(zi_ref, sp_ref, dp_ref, o_ref):
    den = dp_ref[:, 0:1] + dp_ref[:, 1:2]
    den = jnp.where(den > 0, den, 1.0)
    o_ref[...] = jnp.maximum(zi_ref[...] + (sp_ref[0] + sp_ref[1]) / den, 0.0)


def _combine(zi, sp, dp):
    return pl.pallas_call(
        _combine_body,
        grid=(N // _BM,),
        in_specs=[
            pl.BlockSpec((_BM, H), lambda i: (i, 0)),
            pl.BlockSpec((2, _BM, H), lambda i: (0, i, 0)),
            pl.BlockSpec((_BM, 2), lambda i: (i, 0)),
        ],
        out_specs=pl.BlockSpec((_BM, H), lambda i: (i, 0)),
        out_shape=jax.ShapeDtypeStruct((N, H), jnp.float32),
    )(zi, sp, dp)


@functools.partial(
    pl.kernel,
    out_type=[
        jax.ShapeDtypeStruct((_NC, N, H), jnp.float32),
        jax.ShapeDtypeStruct((_NC, 1, _NS * 640), jnp.float32),
    ],
    mesh=_SC_MESH,
    compiler_params=_SC_PARAMS,
    scratch_types=[
        pltpu.VMEM((_EPAD,), jnp.int32),
        pltpu.VMEM((_EPAD,), jnp.int32),
        pltpu.VMEM((_EPAD,), jnp.float32),
        pltpu.VMEM((_L,), jnp.float32),
        pltpu.VMEM_SHARED((N, H), jnp.float32),
        pltpu.VMEM_SHARED((N,), jnp.float32),
        pltpu.VMEM_SHARED((N,), jnp.float32),
        pltpu.VMEM_SHARED((N,), jnp.float32),
        pltpu.SemaphoreType.DMA((_NB1,)),
        pltpu.SemaphoreType.DMA((_NB1,)),
        pltpu.SemaphoreType.DMA((_NB1,)),
        pltpu.SemaphoreType.DMA((_NB1,)),
        pltpu.SemaphoreType.DMA((_NB2,)),
        pltpu.SemaphoreType.DMA((_NB2,)),
    ],
)
def _edge_pass(z_hbm, src_hbm, dst_hbm, ea_hbm, as_hbm, ad_hbm, coef_hbm,
               s_out, den_out,
               src_v, dst_v, w_v, coef_v, s_sp, den_sp, as_sp, ad_sp,
               asem, bsem, esem, dsem, gsem, ssem):
    c = lax.axis_index("c")
    s = lax.axis_index("s")
    wid = c * _NS + s
    ebase = wid * _EPT

    pltpu.sync_copy(src_hbm.at[pl.ds(ebase, _EPT)],
                    src_v.at[pl.ds(0, _EPT)])
    pltpu.sync_copy(dst_hbm.at[pl.ds(ebase, _EPT)],
                    dst_v.at[pl.ds(0, _EPT)])
    pltpu.sync_copy(coef_hbm, coef_v)

    zero = jnp.zeros((_L,), jnp.float32)
    izero = jnp.zeros((_L,), jnp.int32)
    for j in range((_EPAD - _EPT) // _L):
        src_v[pl.ds(_EPT + j * _L, _L)] = izero
        dst_v[pl.ds(_EPT + j * _L, _L)] = izero
        w_v[pl.ds(_EPT + j * _L, _L)] = zero

    row0 = s * _RPT

    def _phase0(zbuf, dzero):
        def _zrow(r, _):
            for j in range(H // _L):
                zbuf[r, pl.ds(j * _L, _L)] = zero
            return 0

        lax.fori_loop(0, _RCH, _zrow, 0)
        for j in range(640 // _L):
            dzero[pl.ds(j * _L, _L)] = zero

        for k in range(_RPT // _RCH):
            pltpu.sync_copy(zbuf, s_sp.at[pl.ds(row0 + k * _RCH, _RCH)])

        @pl.when(s < _NS - 1)
        def _():
            pltpu.sync_copy(dzero.at[pl.ds(0, _RPT)],
                            den_sp.at[pl.ds(row0, _RPT)])

        @pl.when(s == _NS - 1)
        def _():
            pltpu.sync_copy(zbuf.at[pl.ds(0, _L)], s_sp.at[pl.ds(9984, _L)])
            pltpu.sync_copy(dzero, den_sp.at[pl.ds(row0, 640)])

        @pl.when(s == 0)
        def _():
            pltpu.sync_copy(as_hbm, as_sp)

        @pl.when(s == 1)
        def _():
            pltpu.sync_copy(ad_hbm, ad_sp)

    pl.run_scoped(
        _phase0,
        pltpu.VMEM((_RCH, H), jnp.float32),
        pltpu.VMEM((640,), jnp.float32),
    )
    plsc.subcore_barrier()

    def _phase1(aring, bring, ering, wbuf, dstage):
        coefv = coef_v[...]

        for b in range(_NB1):
            srcv0 = src_v[pl.ds(b * _L, _L)]
            dstv0 = dst_v[pl.ds(b * _L, _L)]
            pltpu.async_copy(as_sp.at[srcv0], aring.at[b], asem.at[b])
            pltpu.async_copy(ad_sp.at[dstv0], bring.at[b], bsem.at[b])
            pltpu.async_copy(ea_hbm.at[pl.ds(ebase + b * _L, _L)],
                             ering.at[b], esem.at[b])

        def _outer(t, _):
            for b in range(_NB1):
                g = t * _NB1 + b
                srcv = src_v[pl.ds(g * _L, _L)]
                dstv = dst_v[pl.ds(g * _L, _L)]
                pltpu.make_async_copy(as_sp.at[srcv], aring.at[b],
                                      asem.at[b]).wait()
                pltpu.make_async_copy(ad_sp.at[dstv], bring.at[b],
                                      bsem.at[b]).wait()
                pltpu.make_async_copy(ea_hbm.at[pl.ds(ebase, _L)],
                                      ering.at[b], esem.at[b]).wait()
                x = aring[b, ...] + bring[b, ...] + coefv * ering[b, ...]
                x = jnp.where(x > 0, x, 0.01 * x)
                w = jnp.exp(x)
                w_v[pl.ds(g * _L, _L)] = w

                @pl.when(t > 0)
                def _():
                    pltpu.make_async_copy(wbuf.at[b], den_sp.at[dstv],
                                          dsem.at[b]).wait()

                wbuf[b, ...] = w

                @pl.when(t < _T1 - 1)
                def _():
                    srcv2 = src_v[pl.ds((g + _NB1) * _L, _L)]
                    dstv2 = dst_v[pl.ds((g + _NB1) * _L, _L)]
                    pltpu.async_copy(as_sp.at[srcv2], aring.at[b],
                                     asem.at[b])
                    pltpu.async_copy(ad_sp.at[dstv2], bring.at[b],
                                     bsem.at[b])
                    pltpu.async_copy(
                        ea_hbm.at[pl.ds(ebase + (g + _NB1) * _L, _L)],
                        ering.at[b], esem.at[b])

                pltpu.async_copy(wbuf.at[b], den_sp.at[dstv], dsem.at[b],
                                 add=True)
            return 0

        lax.fori_loop(0, _T1, _outer, 0)

        dstv0 = dst_v[pl.ds(0, _L)]
        for b in range(_NB1):
            pltpu.make_async_copy(wbuf.at[b], den_sp.at[dstv0],
                                  dsem.at[b]).wait()

        plsc.subcore_barrier()

        @pl.when(s < _NS - 1)
        def _():
            pltpu.sync_copy(den_sp.at[pl.ds(row0, _RPT)],
                            dstage.at[pl.ds(0, _RPT)])

        @pl.when(s == _NS - 1)
        def _():
            pltpu.sync_copy(den_sp.at[pl.ds(row0, 640)], dstage)

        pltpu.sync_copy(dstage, den_out.at[c, 0, pl.ds(s * 640, 640)])

    pl.run_scoped(
        _phase1,
        pltpu.VMEM((_NB1, _L), jnp.float32),
        pltpu.VMEM((_NB1, _L), jnp.float32),
        pltpu.VMEM((_NB1, _L), jnp.float32),
        pltpu.VMEM((_NB1, _L), jnp.float32),
        pltpu.VMEM((640,), jnp.float32),
    )

    def _phase2(rbuf, obuf):
        for b in range(_NB2):
            srcv0 = src_v[pl.ds(b * _L, _L)]
            pltpu.async_copy(z_hbm.at[srcv0], rbuf.at[b], gsem.at[b])

        def _outer(t, _):
            for b in range(_NB2):
                g = t * _NB2 + b
                srcv = src_v[pl.ds(g * _L, _L)]
                dstv = dst_v[pl.ds(g * _L, _L)]
                wv = w_v[pl.ds(g * _L, _L)]
                pltpu.make_async_copy(z_hbm.at[srcv], rbuf.at[b],
                                      gsem.at[b]).wait()

                @pl.when(t > 0)
                def _():
                    pltpu.make_async_copy(obuf.at[b], s_sp.at[dstv],
                                          ssem.at[b]).wait()

                for i in range(_L):
                    wvi = jnp.full((_L,), wv[i])
                    for j in range(H // _L):
                        obuf[b, i, pl.ds(j * _L, _L)] = (
                            rbuf[b, i, pl.ds(j * _L, _L)] * wvi)

                @pl.when(t < _T2 - 1)
                def _():
                    srcv2 = src_v[pl.ds((g + _NB2) * _L, _L)]
                    pltpu.async_copy(z_hbm.at[srcv2], rbuf.at[b],
                                     gsem.at[b])

                pltpu.async_copy(obuf.at[b], s_sp.at[dstv], ssem.at[b],
                                 add=True)
            return 0

        lax.fori_loop(0, _T2, _outer, 0)

        dstv0 = dst_v[pl.ds(0, _L)]
        for b in range(_NB2):
            pltpu.make_async_copy(obuf.at[b], s_sp.at[dstv0],
                                  ssem.at[b]).wait()

    pl.run_scoped(
        _phase2,
        pltpu.VMEM((_NB2, _L, H), jnp.float32),
        pltpu.VMEM((_NB2, _L, H), jnp.float32),
    )
    plsc.subcore_barrier()

    def _phase3(cbuf):
        for k in range(_RPT // _RCH):
            pltpu.sync_copy(s_sp.at[pl.ds(row0 + k * _RCH, _RCH)], cbuf)
            pltpu.sync_copy(cbuf, s_out.at[c, pl.ds(row0 + k * _RCH, _RCH)])

        @pl.when(s == _NS - 1)
        def _():
            pltpu.sync_copy(s_sp.at[pl.ds(9984, _L)], cbuf.at[pl.ds(0, _L)])
            pltpu.sync_copy(cbuf.at[pl.ds(0, _L)],
                            s_out.at[c, pl.ds(9984, _L)])

    pl.run_scoped(_phase3, pltpu.VMEM((_RCH, H), jnp.float32))


def _weights(W0, W1, W2, Wa):
    wa_s = Wa[0, :H]
    wa_d = Wa[0, H:2 * H]
    coef = W0[0, 0] * Wa[0, 2 * H]
    u = jnp.stack([W1.T @ wa_s, W1.T @ wa_d], axis=1)
    return W1.T, W2.T, u, jnp.full((_L,), coef, jnp.float32)


def _den_merge(dpart):
    d = dpart.reshape(_NC, _NS, 640)
    head = d[:, :_NS - 1, :_RPT].reshape(_NC, (_NS - 1) * _RPT)
    tail = d[:, _NS - 1, :]
    return jnp.concatenate([head, tail], axis=1).T


def kernel(attr, edge_attr, edge_index, W0_1, W1_1, W2_1, Wa_1,
           W0_2, W1_2, W2_2, Wa_2):
    src = edge_index[0].astype(jnp.int32)
    dst = edge_index[1].astype(jnp.int32)
    ea = edge_attr[:, 0]

    wz1, wi1, u1, coef1 = _weights(W0_1, W1_1, W2_1, Wa_1)
    wz2, wi2, u2, coef2 = _weights(W0_2, W1_2, W2_2, Wa_2)

    z1, zi1, a1 = _mm4(attr, wz1, wi1, u1)
    sp1, dp1 = _edge_pass(z1, src, dst, ea, a1[:, 0], a1[:, 1], coef1)

    z2, zi2, a2 = _combine_mm4(zi1, sp1, _den_merge(dp1), wz2, wi2, u2)
    sp2, dp2 = _edge_pass(z2, src, dst, ea, a2[:, 0], a2[:, 1], coef2)

    return _combine(zi2, sp2, _den_merge(dp2))


# speedup vs baseline: 2.5188x; 2.5188x over previous
import functools
import jax
import jax.numpy as jnp
from jax import lax
from jax.experimental import pallas as pl
from jax.experimental.pallas import tpu as pltpu
from jax.experimental.pallas import tpu_sc as plsc

N = 10000
D = 128
H = 128
E = 320000

_NC = 2
_NS = 16
_NW = _NC * _NS
_L = 16

_EPT = E // _NW
_NB = 5
_GPT = _EPT // _L
_TOUT = _GPT // _NB
_WW = _NB * _L
_RPT = 624
_RCH = 24

_BM = 1000

_SC_PARAMS = pltpu.CompilerParams(needs_layout_passes=False)
_SC_MESH = plsc.VectorSubcoreMesh(core_axis_name="c", subcore_axis_name="s")


def _mm4_body(x_ref, wz_ref, wi_ref, u_ref, z_ref, zi_ref, a_ref):
    x = x_ref[...]
    z_ref[...] = jnp.dot(x, wz_ref[...], preferred_element_type=jnp.float32)
    zi_ref[...] = jnp.dot(x, wi_ref[...], preferred_element_type=jnp.float32)
    a_ref[...] = jnp.dot(x, u_ref[...], preferred_element_type=jnp.float32)


_MM4_OUT = [
    jax.ShapeDtypeStruct((N, H), jnp.float32),
    jax.ShapeDtypeStruct((N, H), jnp.float32),
    jax.ShapeDtypeStruct((N, 2), jnp.float32),
]
_MM4_OUT_SPECS = [
    pl.BlockSpec((_BM, H), lambda i: (i, 0)),
    pl.BlockSpec((_BM, H), lambda i: (i, 0)),
    pl.BlockSpec((_BM, 2), lambda i: (i, 0)),
]


def _mm4(x, wz, wi, u):
    k = x.shape[1]
    return pl.pallas_call(
        _mm4_body,
        grid=(N // _BM,),
        in_specs=[
            pl.BlockSpec((_BM, k), lambda i: (i, 0)),
            pl.BlockSpec((k, H), lambda i: (0, 0)),
            pl.BlockSpec((k, H), lambda i: (0, 0)),
            pl.BlockSpec((k, 2), lambda i: (0, 0)),
        ],
        out_specs=_MM4_OUT_SPECS,
        out_shape=_MM4_OUT,
    )(x, wz, wi, u)


def _combine_mm4_body(zi_ref, sp_ref, dp_ref, wz_ref, wi_ref, u_ref,
                      z_ref, zo_ref, a_ref):
    den = dp_ref[:, 0:1] + dp_ref[:, 1:2]
    den = jnp.where(den > 0, den, 1.0)
    h = jnp.maximum(zi_ref[...] + (sp_ref[0] + sp_ref[1]) / den, 0.0)
    z_ref[...] = jnp.dot(h, wz_ref[...], preferred_element_type=jnp.float32)
    zo_ref[...] = jnp.dot(h, wi_ref[...], preferred_element_type=jnp.float32)
    a_ref[...] = jnp.dot(h, u_ref[...], preferred_element_type=jnp.float32)


def _combine_mm4(zi, sp, dp, wz, wi, u):
    return pl.pallas_call(
        _combine_mm4_body,
        grid=(N // _BM,),
        in_specs=[
            pl.BlockSpec((_BM, H), lambda i: (i, 0)),
            pl.BlockSpec((2, _BM, H), lambda i: (0, i, 0)),
            pl.BlockSpec((_BM, 2), lambda i: (i, 0)),
            pl.BlockSpec((H, H), lambda i: (0, 0)),
            pl.BlockSpec((H, H), lambda i: (0, 0)),
            pl.BlockSpec((H, 2), lambda i: (0, 0)),
        ],
        out_specs=_MM4_OUT_SPECS,
        out_shape=_MM4_OUT,
    )(zi, sp, dp, wz, wi, u)


def _combine_body(zi_ref, sp_ref, dp_ref, o_ref):
    den = dp_ref[:, 0:1] + dp_ref[:, 1:2]
    den = jnp.where(den > 0, den, 1.0)
    o_ref[...] = jnp.maximum(zi_ref[...] + (sp_ref[0] + sp_ref[1]) / den, 0.0)


def _combine(zi, sp, dp):
    return pl.pallas_call(
        _combine_body,
        grid=(N // _BM,),
        in_specs=[
            pl.BlockSpec((_BM, H), lambda i: (i, 0)),
            pl.BlockSpec((2, _BM, H), lambda i: (0, i, 0)),
            pl.BlockSpec((_BM, 2), lambda i: (i, 0)),
        ],
        out_specs=pl.BlockSpec((_BM, H), lambda i: (i, 0)),
        out_shape=jax.ShapeDtypeStruct((N, H), jnp.float32),
    )(zi, sp, dp)


@functools.partial(
    pl.kernel,
    out_type=[
        jax.ShapeDtypeStruct((E,), jnp.float32),
        jax.ShapeDtypeStruct((_NC, 1, N), jnp.float32),
    ],
    mesh=_SC_MESH,
    compiler_params=_SC_PARAMS,
    scratch_types=[
        pltpu.VMEM((_EPT,), jnp.int32),
        pltpu.VMEM((_EPT,), jnp.int32),
        pltpu.VMEM((_EPT,), jnp.float32),
        pltpu.VMEM((_EPT,), jnp.float32),
        pltpu.VMEM((N,), jnp.float32),
        pltpu.VMEM((N,), jnp.float32),
        pltpu.VMEM((_L,), jnp.float32),
        pltpu.VMEM((_NB, _L), jnp.float32),
        pltpu.VMEM((1, N), jnp.float32),
        pltpu.VMEM_SHARED((N,), jnp.float32),
        pltpu.SemaphoreType.DMA((_NB,)),
    ],
)
def _edge_weights(src_hbm, dst_hbm, ea_hbm, as_hbm, ad_hbm, coef_hbm,
                  w_out, den_out,
                  src_v, dst_v, ea_v, w_v, as_v, ad_v, coef_v,
                  wbuf, dden, den_sp, dsem):
    c = lax.axis_index("c")
    s = lax.axis_index("s")
    wid = c * _NS + s
    ebase = wid * _EPT

    pltpu.sync_copy(src_hbm.at[pl.ds(ebase, _EPT)], src_v)
    pltpu.sync_copy(dst_hbm.at[pl.ds(ebase, _EPT)], dst_v)
    pltpu.sync_copy(ea_hbm.at[pl.ds(ebase, _EPT)], ea_v)
    pltpu.sync_copy(as_hbm, as_v)
    pltpu.sync_copy(ad_hbm, ad_v)
    pltpu.sync_copy(coef_hbm, coef_v)

    zero = jnp.zeros((_L,), jnp.float32)

    @pl.when(s == 0)
    def _():
        def _zden(r, _):
            dden[0, pl.ds(r * _L, _L)] = zero
            return 0
        lax.fori_loop(0, N // _L, _zden, 0)
        pltpu.sync_copy(dden.at[0], den_sp)

    plsc.subcore_barrier()

    coefv = coef_v[...]

    def _outer(t, _):
        for b in range(_NB):
            g = t * _NB + b
            srcv = src_v[pl.ds(g * _L, _L)]
            dstv = dst_v[pl.ds(g * _L, _L)]
            tv = ea_v[pl.ds(g * _L, _L)]
            x = (plsc.load_gather(as_v, [srcv])
                 + plsc.load_gather(ad_v, [dstv]) + coefv * tv)
            x = jnp.where(x > 0, x, 0.01 * x)
            w = jnp.exp(x)
            w_v[pl.ds(g * _L, _L)] = w

            @pl.when(t > 0)
            def _():
                pltpu.make_async_copy(wbuf.at[b], den_sp.at[dstv],
                                      dsem.at[b]).wait()

            wbuf[b, ...] = w
            pltpu.async_copy(wbuf.at[b], den_sp.at[dstv], dsem.at[b],
                             add=True)
        return 0

    lax.fori_loop(0, _TOUT, _outer, 0)

    dstv0 = dst_v[pl.ds(0, _L)]
    for b in range(_NB):
        pltpu.make_async_copy(wbuf.at[b], den_sp.at[dstv0],
                              dsem.at[b]).wait()

    pltpu.sync_copy(w_v, w_out.at[pl.ds(ebase, _EPT)])

    plsc.subcore_barrier()

    @pl.when(s == 0)
    def _():
        pltpu.sync_copy(den_sp, dden.at[0])
        pltpu.sync_copy(dden, den_out.at[c])


@functools.partial(
    pl.kernel,
    out_type=jax.ShapeDtypeStruct((_NC, N, H), jnp.float32),
    mesh=_SC_MESH,
    compiler_params=_SC_PARAMS,
    scratch_types=[
        pltpu.VMEM((_EPT,), jnp.int32),
        pltpu.VMEM((_EPT,), jnp.int32),
        pltpu.VMEM((2 * _WW,), jnp.float32),
        pltpu.VMEM((_NB, _L, H), jnp.float32),
        pltpu.VMEM((_NB, _L, H), jnp.float32),
        pltpu.VMEM((_RCH, H), jnp.float32),
        pltpu.VMEM_SHARED((N, H), jnp.float32),
        pltpu.SemaphoreType.DMA((2,)),
        pltpu.SemaphoreType.DMA((_NB,)),
        pltpu.SemaphoreType.DMA((_NB,)),
    ],
)
def _edge_scatter(z_hbm, src_hbm, dst_hbm, w_hbm, s_out,
                  src_v, dst_v, wring, rbuf, obuf, stage,
                  s_sp, wsem, gsem, ssem):
    c = lax.axis_index("c")
    s = lax.axis_index("s")
    wid = c * _NS + s
    ebase = wid * _EPT

    pltpu.sync_copy(src_hbm.at[pl.ds(ebase, _EPT)], src_v)
    pltpu.sync_copy(dst_hbm.at[pl.ds(ebase, _EPT)], dst_v)

    zero = jnp.zeros((_L,), jnp.float32)

    def _zrow(r, _):
        for j in range(H // _L):
            stage[r, pl.ds(j * _L, _L)] = zero
        return 0

    lax.fori_loop(0, _RCH, _zrow, 0)

    row0 = s * _RPT
    for k in range(_RPT // _RCH):
        pltpu.sync_copy(stage, s_sp.at[pl.ds(row0 + k * _RCH, _RCH)])

    @pl.when(s == _NS - 1)
    def _():
        pltpu.sync_copy(stage.at[pl.ds(0, _L)], s_sp.at[pl.ds(9984, _L)])

    plsc.subcore_barrier()

    for b in range(_NB):
        srcv0 = src_v[pl.ds(b * _L, _L)]
        pltpu.async_copy(z_hbm.at[srcv0], rbuf.at[b], gsem.at[b])
    pltpu.async_copy(w_hbm.at[pl.ds(ebase, _WW)],
                     wring.at[pl.ds(0, _WW)], wsem.at[0])

    def _outer(t, _):
        p = lax.rem(t, 2)
        woff = p * _WW
        pltpu.make_async_copy(w_hbm.at[pl.ds(ebase, _WW)],
                              wring.at[pl.ds(woff, _WW)], wsem.at[p]).wait()

        @pl.when(t < _TOUT - 1)
        def _():
            pltpu.async_copy(
                w_hbm.at[pl.ds(ebase + (t + 1) * _WW, _WW)],
                wring.at[pl.ds((1 - p) * _WW, _WW)], wsem.at[1 - p])

        for b in range(_NB):
            g = t * _NB + b
            srcv = src_v[pl.ds(g * _L, _L)]
            dstv = dst_v[pl.ds(g * _L, _L)]
            wv = wring[pl.ds(woff + b * _L, _L)]
            pltpu.make_async_copy(z_hbm.at[srcv], rbuf.at[b],
                                  gsem.at[b]).wait()

            @pl.when(t > 0)
            def _():
                pltpu.make_async_copy(obuf.at[b], s_sp.at[dstv],
                                      ssem.at[b]).wait()

            for i in range(_L):
                wvi = jnp.full((_L,), wv[i])
                for j in range(H // _L):
                    obuf[b, i, pl.ds(j * _L, _L)] = (
                        rbuf[b, i, pl.ds(j * _L, _L)] * wvi)

            @pl.when(t < _TOUT - 1)
            def _():
                srcv2 = src_v[pl.ds((g + _NB) * _L, _L)]
                pltpu.async_copy(z_hbm.at[srcv2], rbuf.at[b], gsem.at[b])

            pltpu.async_copy(obuf.at[b], s_sp.at[dstv], ssem.at[b],
                             add=True)
        return 0

    lax.fori_loop(0, _TOUT, _outer, 0)

    dstv0 = dst_v[pl.ds(0, _L)]
    for b in range(_NB):
        pltpu.make_async_copy(obuf.at[b], s_sp.at[dstv0],
                              ssem.at[b]).wait()

    plsc.subcore_barrier()

    for k in range(_RPT // _RCH):
        pltpu.sync_copy(s_sp.at[pl.ds(row0 + k * _RCH, _RCH)], stage)
        pltpu.sync_copy(stage, s_out.at[c, pl.ds(row0 + k * _RCH, _RCH)])

    @pl.when(s == _NS - 1)
    def _():
        pltpu.sync_copy(s_sp.at[pl.ds(9984, _L)], stage.at[pl.ds(0, _L)])
        pltpu.sync_copy(stage.at[pl.ds(0, _L)], s_out.at[c, pl.ds(9984, _L)])


def _weights(W0, W1, W2, Wa):
    wa_s = Wa[0, :H]
    wa_d = Wa[0, H:2 * H]
    coef = W0[0, 0] * Wa[0, 2 * H]
    u = jnp.stack([W1.T @ wa_s, W1.T @ wa_d], axis=1)
    return W1.T, W2.T, u, jnp.full((_L,), coef, jnp.float32)


def kernel(attr, edge_attr, edge_index, W0_1, W1_1, W2_1, Wa_1,
           W0_2, W1_2, W2_2, Wa_2):
    src = edge_index[0].astype(jnp.int32)
    dst = edge_index[1].astype(jnp.int32)
    ea = edge_attr[:, 0]

    wz1, wi1, u1, coef1 = _weights(W0_1, W1_1, W2_1, Wa_1)
    wz2, wi2, u2, coef2 = _weights(W0_2, W1_2, W2_2, Wa_2)

    z1, zi1, a1 = _mm4(attr, wz1, wi1, u1)
    w1, dp1 = _edge_weights(src, dst, ea, a1[:, 0], a1[:, 1], coef1)
    sp1 = _edge_scatter(z1, src, dst, w1)

    z2, zi2, a2 = _combine_mm4(zi1, sp1, dp1.reshape(_NC, N).T, wz2, wi2, u2)
    w2, dp2 = _edge_weights(src, dst, ea, a2[:, 0], a2[:, 1], coef2)
    sp2 = _edge_scatter(z2, src, dst, w2)

    return _combine(zi2, sp2, dp2.reshape(_NC, N).T)
